# Initial kernel scaffold; baseline (speedup 1.0000x reference)
import jax
import jax.numpy as jnp
from jax.experimental import pallas as pl


def kernel(x, edge_index, batch, W1, b1, W2, b2, Wl, bl):
    raise NotImplementedError("write your pallas kernel here")


# speedup vs baseline: 11.2381x; 11.2381x over previous
import functools

import jax
import jax.numpy as jnp
from jax import lax
from jax.experimental import pallas as pl
from jax.experimental.pallas import tpu as pltpu
from jax.experimental.pallas import tpu_sc as plsc

N = 10000
E = 320000
IN_CH = 128
HID = 64
G = 128

NC = 2
NS = 16
NW = NC * NS

NPAD = 10240
EPAD = 327680
EPW = EPAD // NW
CH = 128
NCHUNK = EPW // CH
SLAB = NPAD // NS

BN = 512
NBLK = NPAD // BN

_mesh = plsc.VectorSubcoreMesh(core_axis_name="c", subcore_axis_name="s")


def _deg_body(dst_hbm, out_hbm, dst_v, deg_v):
    c = lax.axis_index("c")
    s = lax.axis_index("s")
    wid = c * NS + s

    def zero(i, carry):
        deg_v[pl.ds(i * 16, 16)] = jnp.zeros((16,), jnp.float32)
        return carry

    lax.fori_loop(0, NPAD // 16, zero, 0)

    pltpu.sync_copy(dst_hbm.at[pl.ds(wid * EPW, EPW)], dst_v)

    ones = jnp.ones((16,), jnp.float32)

    def acc(j, carry):
        dvec = dst_v[pl.ds(j * 16, 16)]
        plsc.addupdate_scatter(deg_v, [dvec], ones)
        return carry

    lax.fori_loop(0, EPW // 16, acc, 0)

    pltpu.sync_copy(deg_v, out_hbm.at[wid])


_deg_call = pl.kernel(
    _deg_body,
    out_type=jax.ShapeDtypeStruct((NW, NPAD), jnp.float32),
    mesh=_mesh,
    compiler_params=pltpu.CompilerParams(needs_layout_passes=False),
    scratch_types=[
        pltpu.VMEM((EPW,), jnp.int32),
        pltpu.VMEM((NPAD,), jnp.float32),
    ],
)


def _agg_body(src_hbm, dst_hbm, hp_hbm, zero_hbm, out_hbm,
              sidx_v, didx_v, rows_v, acc_sh, sem):
    c = lax.axis_index("c")
    s = lax.axis_index("s")
    wid = c * NS + s

    pltpu.sync_copy(zero_hbm.at[pl.ds(s * SLAB, SLAB)],
                    acc_sh.at[pl.ds(s * SLAB, SLAB)])
    plsc.subcore_barrier()

    def chunk(k, carry):
        base = wid * EPW + k * CH
        pltpu.sync_copy(src_hbm.at[pl.ds(base, CH)], sidx_v)
        pltpu.sync_copy(dst_hbm.at[pl.ds(base, CH)], didx_v)
        pltpu.async_copy(hp_hbm.at[sidx_v], rows_v, sem).wait()
        pltpu.sync_copy(rows_v, acc_sh.at[didx_v], add=True)
        return carry

    lax.fori_loop(0, NCHUNK, chunk, 0)

    plsc.subcore_barrier()
    pltpu.sync_copy(acc_sh.at[pl.ds(s * SLAB, SLAB)],
                    out_hbm.at[c, pl.ds(s * SLAB, SLAB)])


_agg_call = pl.kernel(
    _agg_body,
    out_type=jax.ShapeDtypeStruct((NC, NPAD, HID), jnp.float32),
    mesh=_mesh,
    compiler_params=pltpu.CompilerParams(needs_layout_passes=False,
                                         use_tc_tiling_on_sc=False),
    scratch_types=[
        pltpu.VMEM((CH,), jnp.int32),
        pltpu.VMEM((CH,), jnp.int32),
        pltpu.VMEM((CH, HID), jnp.float32),
        pltpu.VMEM_SHARED((NPAD, HID), jnp.float32),
        pltpu.SemaphoreType.DMA,
    ],
)


def _tca_body(parts_ref, x_ref, w1_ref, hp_ref, dis_ref):
    deg = jnp.sum(parts_ref[...], axis=0) + 1.0
    dis = lax.rsqrt(deg)
    h = jnp.dot(x_ref[...], w1_ref[...], preferred_element_type=jnp.float32)
    hp_ref[...] = h * dis[:, None]
    dis_ref[...] = dis[:, None]


_tca_call = pl.pallas_call(
    _tca_body,
    grid=(NBLK,),
    in_specs=[
        pl.BlockSpec((NW, BN), lambda i: (0, i)),
        pl.BlockSpec((BN, IN_CH), lambda i: (i, 0)),
        pl.BlockSpec((IN_CH, HID), lambda i: (0, 0)),
    ],
    out_specs=[
        pl.BlockSpec((BN, HID), lambda i: (i, 0)),
        pl.BlockSpec((BN, 1), lambda i: (i, 0)),
    ],
    out_shape=[
        jax.ShapeDtypeStruct((NPAD, HID), jnp.float32),
        jax.ShapeDtypeStruct((NPAD, 1), jnp.float32),
    ],
)


def _tcb_body(acc_ref, hp_ref, dis_ref, b1_ref, w2_ref, hp2_ref):
    a = acc_ref[0] + acc_ref[1]
    h1 = jnp.maximum((a + hp_ref[...]) * dis_ref[...] + b1_ref[...], 0.0)
    h2 = jnp.dot(h1, w2_ref[...], preferred_element_type=jnp.float32)
    hp2_ref[...] = h2 * dis_ref[...]


_tcb_call = pl.pallas_call(
    _tcb_body,
    grid=(NBLK,),
    in_specs=[
        pl.BlockSpec((NC, BN, HID), lambda i: (0, i, 0)),
        pl.BlockSpec((BN, HID), lambda i: (i, 0)),
        pl.BlockSpec((BN, 1), lambda i: (i, 0)),
        pl.BlockSpec((1, HID), lambda i: (0, 0)),
        pl.BlockSpec((HID, HID), lambda i: (0, 0)),
    ],
    out_specs=pl.BlockSpec((BN, HID), lambda i: (i, 0)),
    out_shape=jax.ShapeDtypeStruct((NPAD, HID), jnp.float32),
)


def _tcc_body(acc_ref, hp2_ref, dis_ref, b2_ref, batch_ref, wl_ref, bl_ref,
              out_ref, sum_scr, cnt_scr):
    i = pl.program_id(0)

    @pl.when(i == 0)
    def _():
        sum_scr[...] = jnp.zeros_like(sum_scr)
        cnt_scr[...] = jnp.zeros_like(cnt_scr)

    a = acc_ref[0] + acc_ref[1]
    h2 = jnp.maximum((a + hp2_ref[...]) * dis_ref[...] + b2_ref[...], 0.0)
    b = batch_ref[0, 0]
    gids = lax.broadcasted_iota(jnp.int32, (G, BN), 0)
    onehot = (gids == b[None, :]).astype(jnp.float32)
    sum_scr[...] += jnp.dot(onehot, h2, preferred_element_type=jnp.float32, precision=lax.Precision.HIGHEST)
    cnt_scr[...] += jnp.dot(onehot, jnp.ones((BN, HID), jnp.float32),
                            preferred_element_type=jnp.float32, precision=lax.Precision.HIGHEST)

    @pl.when(i == NBLK - 1)
    def _():
        pooled = sum_scr[...] / jnp.maximum(cnt_scr[...], 1.0)
        out_ref[...] = (jnp.dot(pooled, wl_ref[...],
                                preferred_element_type=jnp.float32)
                        + bl_ref[...])


_tcc_call = pl.pallas_call(
    _tcc_body,
    grid=(NBLK,),
    in_specs=[
        pl.BlockSpec((NC, BN, HID), lambda i: (0, i, 0)),
        pl.BlockSpec((BN, HID), lambda i: (i, 0)),
        pl.BlockSpec((BN, 1), lambda i: (i, 0)),
        pl.BlockSpec((1, HID), lambda i: (0, 0)),
        pl.BlockSpec((1, 1, BN), lambda i: (i, 0, 0)),
        pl.BlockSpec((HID, 1), lambda i: (0, 0)),
        pl.BlockSpec((1, 1), lambda i: (0, 0)),
    ],
    out_specs=pl.BlockSpec((G, 1), lambda i: (0, 0)),
    out_shape=jax.ShapeDtypeStruct((G, 1), jnp.float32),
    scratch_shapes=[
        pltpu.VMEM((G, HID), jnp.float32),
        pltpu.VMEM((G, HID), jnp.float32),
    ],
)


@jax.jit
def kernel(x, edge_index, batch, W1, b1, W2, b2, Wl, bl):
    src = edge_index[0].astype(jnp.int32)
    dst = edge_index[1].astype(jnp.int32)
    src_p = jnp.concatenate([src, jnp.zeros((EPAD - E,), jnp.int32)])
    dst_p = jnp.concatenate([dst, jnp.full((EPAD - E,), N, jnp.int32)])
    x_p = jnp.concatenate([x, jnp.zeros((NPAD - N, IN_CH), jnp.float32)])
    batch_p = jnp.concatenate(
        [batch.astype(jnp.int32), jnp.full((NPAD - N,), G, jnp.int32)]
    ).reshape(NBLK, 1, BN)
    zeros_nod = jnp.zeros((NPAD, HID), jnp.float32)

    deg_parts = _deg_call(dst_p)
    hp, dis = _tca_call(deg_parts, x_p, W1)
    acc1 = _agg_call(src_p, dst_p, hp, zeros_nod)
    hp2 = _tcb_call(acc1, hp, dis, b1.reshape(1, HID), W2)
    acc2 = _agg_call(src_p, dst_p, hp2, zeros_nod)
    out = _tcc_call(acc2, hp2, dis, b2.reshape(1, HID), batch_p,
                    Wl, bl.reshape(1, 1))
    return out


# speedup vs baseline: 14.7333x; 1.3110x over previous
import functools

import jax
import jax.numpy as jnp
from jax import lax
from jax.experimental import pallas as pl
from jax.experimental.pallas import tpu as pltpu
from jax.experimental.pallas import tpu_sc as plsc

N = 10000
E = 320000
IN_CH = 128
HID = 64
G = 128

NC = 2
NS = 16
NW = NC * NS

NPAD = 10240
EPAD = 327680
EPW = EPAD // NW
CH = 128
NCHUNK = EPW // CH
SLAB = NPAD // NS

BN = 512
NBLK = NPAD // BN

_mesh = plsc.VectorSubcoreMesh(core_axis_name="c", subcore_axis_name="s")


def _deg_body(dst_hbm, out_hbm, dst_v, deg_v):
    c = lax.axis_index("c")
    s = lax.axis_index("s")
    wid = c * NS + s

    def zero(i, carry):
        deg_v[pl.ds(i * 16, 16)] = jnp.zeros((16,), jnp.float32)
        return carry

    lax.fori_loop(0, NPAD // 16, zero, 0)

    pltpu.sync_copy(dst_hbm.at[pl.ds(wid * EPW, EPW)], dst_v)

    ones = jnp.ones((16,), jnp.float32)

    def acc(j, carry):
        dvec = dst_v[pl.ds(j * 16, 16)]
        plsc.addupdate_scatter(deg_v, [dvec], ones)
        return carry

    lax.fori_loop(0, EPW // 16, acc, 0)

    pltpu.sync_copy(deg_v, out_hbm.at[wid])


_deg_call = pl.kernel(
    _deg_body,
    out_type=jax.ShapeDtypeStruct((NW, NPAD), jnp.float32),
    mesh=_mesh,
    compiler_params=pltpu.CompilerParams(needs_layout_passes=False),
    scratch_types=[
        pltpu.VMEM((EPW,), jnp.int32),
        pltpu.VMEM((NPAD,), jnp.float32),
    ],
)


def _agg_body(src_hbm, dst_hbm, hp_hbm, zero_hbm, out_hbm,
              sidx_v, didx_v, rows0, rows1, acc_sh, sg0, sg1, ss0, ss1):
    c = lax.axis_index("c")
    s = lax.axis_index("s")
    wid = c * NS + s

    pltpu.sync_copy(zero_hbm.at[pl.ds(s * SLAB, SLAB)],
                    acc_sh.at[pl.ds(s * SLAB, SLAB)])
    pltpu.sync_copy(src_hbm.at[wid], sidx_v)
    pltpu.sync_copy(dst_hbm.at[wid], didx_v)
    plsc.subcore_barrier()

    pltpu.async_copy(hp_hbm.at[sidx_v.at[0]], rows0, sg0)
    pltpu.async_copy(hp_hbm.at[sidx_v.at[1]], rows1, sg1)

    def pair(p, carry):
        k = 2 * p
        pltpu.make_async_copy(hp_hbm.at[sidx_v.at[k]], rows0, sg0).wait()
        pltpu.async_copy(rows0, acc_sh.at[didx_v.at[k]], ss0, add=True)
        pltpu.make_async_copy(hp_hbm.at[sidx_v.at[k]], rows1, sg1).wait()
        pltpu.async_copy(rows1, acc_sh.at[didx_v.at[k + 1]], ss1, add=True)
        pltpu.make_async_copy(rows0, acc_sh.at[didx_v.at[k]], ss0).wait()
        pltpu.async_copy(hp_hbm.at[sidx_v.at[lax.rem(k + 2, NCHUNK)]],
                         rows0, sg0)
        pltpu.make_async_copy(rows1, acc_sh.at[didx_v.at[k]], ss1).wait()
        pltpu.async_copy(hp_hbm.at[sidx_v.at[lax.rem(k + 3, NCHUNK)]],
                         rows1, sg1)
        return carry

    lax.fori_loop(0, NCHUNK // 2, pair, 0)

    pltpu.make_async_copy(hp_hbm.at[sidx_v.at[0]], rows0, sg0).wait()
    pltpu.make_async_copy(hp_hbm.at[sidx_v.at[1]], rows1, sg1).wait()

    plsc.subcore_barrier()
    pltpu.sync_copy(acc_sh.at[pl.ds(s * SLAB, SLAB)],
                    out_hbm.at[c, pl.ds(s * SLAB, SLAB)])


_agg_call = pl.kernel(
    _agg_body,
    out_type=jax.ShapeDtypeStruct((NC, NPAD, HID), jnp.float32),
    mesh=_mesh,
    compiler_params=pltpu.CompilerParams(needs_layout_passes=False,
                                         use_tc_tiling_on_sc=False),
    scratch_types=[
        pltpu.VMEM((NCHUNK, CH), jnp.int32),
        pltpu.VMEM((NCHUNK, CH), jnp.int32),
        pltpu.VMEM((CH, HID), jnp.float32),
        pltpu.VMEM((CH, HID), jnp.float32),
        pltpu.VMEM_SHARED((NPAD, HID), jnp.float32),
        pltpu.SemaphoreType.DMA,
        pltpu.SemaphoreType.DMA,
        pltpu.SemaphoreType.DMA,
        pltpu.SemaphoreType.DMA,
    ],
)


def _tca_body(parts_ref, x_ref, w1_ref, hp_ref, dis_ref):
    deg = jnp.sum(parts_ref[...], axis=0) + 1.0
    dis = lax.rsqrt(deg)
    h = jnp.dot(x_ref[...], w1_ref[...], preferred_element_type=jnp.float32)
    hp_ref[...] = h * dis[:, None]
    dis_ref[...] = dis[:, None]


_tca_call = pl.pallas_call(
    _tca_body,
    grid=(NBLK,),
    in_specs=[
        pl.BlockSpec((NW, BN), lambda i: (0, i)),
        pl.BlockSpec((BN, IN_CH), lambda i: (i, 0)),
        pl.BlockSpec((IN_CH, HID), lambda i: (0, 0)),
    ],
    out_specs=[
        pl.BlockSpec((BN, HID), lambda i: (i, 0)),
        pl.BlockSpec((BN, 1), lambda i: (i, 0)),
    ],
    out_shape=[
        jax.ShapeDtypeStruct((NPAD, HID), jnp.float32),
        jax.ShapeDtypeStruct((NPAD, 1), jnp.float32),
    ],
)


def _tcb_body(acc_ref, hp_ref, dis_ref, b1_ref, w2_ref, hp2_ref):
    a = acc_ref[0] + acc_ref[1]
    h1 = jnp.maximum((a + hp_ref[...]) * dis_ref[...] + b1_ref[...], 0.0)
    h2 = jnp.dot(h1, w2_ref[...], preferred_element_type=jnp.float32)
    hp2_ref[...] = h2 * dis_ref[...]


_tcb_call = pl.pallas_call(
    _tcb_body,
    grid=(NBLK,),
    in_specs=[
        pl.BlockSpec((NC, BN, HID), lambda i: (0, i, 0)),
        pl.BlockSpec((BN, HID), lambda i: (i, 0)),
        pl.BlockSpec((BN, 1), lambda i: (i, 0)),
        pl.BlockSpec((1, HID), lambda i: (0, 0)),
        pl.BlockSpec((HID, HID), lambda i: (0, 0)),
    ],
    out_specs=pl.BlockSpec((BN, HID), lambda i: (i, 0)),
    out_shape=jax.ShapeDtypeStruct((NPAD, HID), jnp.float32),
)


def _tcc_body(acc_ref, hp2_ref, dis_ref, b2_ref, batch_ref, wl_ref, bl_ref,
              out_ref, sum_scr, cnt_scr):
    i = pl.program_id(0)

    @pl.when(i == 0)
    def _():
        sum_scr[...] = jnp.zeros_like(sum_scr)
        cnt_scr[...] = jnp.zeros_like(cnt_scr)

    a = acc_ref[0] + acc_ref[1]
    h2 = jnp.maximum((a + hp2_ref[...]) * dis_ref[...] + b2_ref[...], 0.0)
    b = batch_ref[0, 0]
    gids = lax.broadcasted_iota(jnp.int32, (G, BN), 0)
    onehot = (gids == b[None, :]).astype(jnp.float32)
    sum_scr[...] += jnp.dot(onehot, h2, preferred_element_type=jnp.float32, precision=lax.Precision.HIGHEST)
    cnt_scr[...] += jnp.dot(onehot, jnp.ones((BN, HID), jnp.float32),
                            preferred_element_type=jnp.float32, precision=lax.Precision.HIGHEST)

    @pl.when(i == NBLK - 1)
    def _():
        pooled = sum_scr[...] / jnp.maximum(cnt_scr[...], 1.0)
        out_ref[...] = (jnp.dot(pooled, wl_ref[...],
                                preferred_element_type=jnp.float32)
                        + bl_ref[...])


_tcc_call = pl.pallas_call(
    _tcc_body,
    grid=(NBLK,),
    in_specs=[
        pl.BlockSpec((NC, BN, HID), lambda i: (0, i, 0)),
        pl.BlockSpec((BN, HID), lambda i: (i, 0)),
        pl.BlockSpec((BN, 1), lambda i: (i, 0)),
        pl.BlockSpec((1, HID), lambda i: (0, 0)),
        pl.BlockSpec((1, 1, BN), lambda i: (i, 0, 0)),
        pl.BlockSpec((HID, 1), lambda i: (0, 0)),
        pl.BlockSpec((1, 1), lambda i: (0, 0)),
    ],
    out_specs=pl.BlockSpec((G, 1), lambda i: (0, 0)),
    out_shape=jax.ShapeDtypeStruct((G, 1), jnp.float32),
    scratch_shapes=[
        pltpu.VMEM((G, HID), jnp.float32),
        pltpu.VMEM((G, HID), jnp.float32),
    ],
)


@jax.jit
def kernel(x, edge_index, batch, W1, b1, W2, b2, Wl, bl):
    src = edge_index[0].astype(jnp.int32)
    dst = edge_index[1].astype(jnp.int32)
    src_p = jnp.concatenate(
        [src, jnp.zeros((EPAD - E,), jnp.int32)]).reshape(NW, NCHUNK, CH)
    dst_p = jnp.concatenate(
        [dst, jnp.full((EPAD - E,), N, jnp.int32)]).reshape(NW, NCHUNK, CH)
    x_p = jnp.concatenate([x, jnp.zeros((NPAD - N, IN_CH), jnp.float32)])
    batch_p = jnp.concatenate(
        [batch.astype(jnp.int32), jnp.full((NPAD - N,), G, jnp.int32)]
    ).reshape(NBLK, 1, BN)
    zeros_nod = jnp.zeros((NPAD, HID), jnp.float32)

    deg_parts = _deg_call(dst_p.reshape(EPAD))
    hp, dis = _tca_call(deg_parts, x_p, W1)
    acc1 = _agg_call(src_p, dst_p, hp, zeros_nod)
    hp2 = _tcb_call(acc1, hp, dis, b1.reshape(1, HID), W2)
    acc2 = _agg_call(src_p, dst_p, hp2, zeros_nod)
    out = _tcc_call(acc2, hp2, dis, b2.reshape(1, HID), batch_p,
                    Wl, bl.reshape(1, 1))
    return out


# speedup vs baseline: 33.3329x; 2.2624x over previous
import functools

import jax
import jax.numpy as jnp
from jax import lax
from jax.experimental import pallas as pl
from jax.experimental.pallas import tpu as pltpu
from jax.experimental.pallas import tpu_sc as plsc

N = 10000
E = 320000
IN_CH = 128
HID = 64
G = 128

NC = 2
NS = 16
NW = NC * NS

NPAD = 10240
EPAD = 327680
EPW = EPAD // NW
CH = 128
NCHUNK = EPW // CH
SLAB = NPAD // NS

BN = 512
NBLK = NPAD // BN

_mesh = plsc.VectorSubcoreMesh(core_axis_name="c", subcore_axis_name="s")


def _deg_body(dst_hbm, out_hbm, dst_v, deg_v):
    c = lax.axis_index("c")
    s = lax.axis_index("s")
    wid = c * NS + s

    def zero(i, carry):
        deg_v[pl.ds(i * 16, 16)] = jnp.zeros((16,), jnp.float32)
        return carry

    lax.fori_loop(0, NPAD // 16, zero, 0)

    pltpu.sync_copy(dst_hbm.at[pl.ds(wid * EPW, EPW)], dst_v)

    ones = jnp.ones((16,), jnp.float32)

    def acc(j, carry):
        dvec = dst_v[pl.ds(j * 16, 16)]
        plsc.addupdate_scatter(deg_v, [dvec], ones)
        return carry

    lax.fori_loop(0, EPW // 16, acc, 0)

    pltpu.sync_copy(deg_v, out_hbm.at[wid])


_deg_call = pl.kernel(
    _deg_body,
    out_type=jax.ShapeDtypeStruct((NW, NPAD), jnp.float32),
    mesh=_mesh,
    compiler_params=pltpu.CompilerParams(needs_layout_passes=False),
    scratch_types=[
        pltpu.VMEM((EPW,), jnp.int32),
        pltpu.VMEM((NPAD,), jnp.float32),
    ],
)


def _agg_body(src_hbm, dst_hbm, hp_hbm, zero_hbm, out_hbm,
              sidx_v, didx_v, rows0, rows1, acc_sh, sg0, sg1, ss0, ss1):
    c = lax.axis_index("c")
    s = lax.axis_index("s")
    wid = c * NS + s

    pltpu.sync_copy(zero_hbm.at[pl.ds(s * SLAB, SLAB)],
                    acc_sh.at[pl.ds(s * SLAB, SLAB)])
    pltpu.sync_copy(src_hbm.at[wid], sidx_v)
    pltpu.sync_copy(dst_hbm.at[wid], didx_v)
    plsc.subcore_barrier()

    pltpu.async_copy(hp_hbm.at[sidx_v.at[0]], rows0, sg0)
    pltpu.async_copy(hp_hbm.at[sidx_v.at[1]], rows1, sg1)

    def pair(p, carry):
        k = 2 * p
        pltpu.make_async_copy(hp_hbm.at[sidx_v.at[k]], rows0, sg0).wait()
        pltpu.async_copy(rows0, acc_sh.at[didx_v.at[k]], ss0, add=True)
        pltpu.make_async_copy(hp_hbm.at[sidx_v.at[k]], rows1, sg1).wait()
        pltpu.async_copy(rows1, acc_sh.at[didx_v.at[k + 1]], ss1, add=True)
        pltpu.make_async_copy(rows0, acc_sh.at[didx_v.at[k]], ss0).wait()
        pltpu.async_copy(hp_hbm.at[sidx_v.at[lax.rem(k + 2, NCHUNK)]],
                         rows0, sg0)
        pltpu.make_async_copy(rows1, acc_sh.at[didx_v.at[k]], ss1).wait()
        pltpu.async_copy(hp_hbm.at[sidx_v.at[lax.rem(k + 3, NCHUNK)]],
                         rows1, sg1)
        return carry

    lax.fori_loop(0, NCHUNK // 2, pair, 0)

    pltpu.make_async_copy(hp_hbm.at[sidx_v.at[0]], rows0, sg0).wait()
    pltpu.make_async_copy(hp_hbm.at[sidx_v.at[1]], rows1, sg1).wait()

    plsc.subcore_barrier()
    pltpu.sync_copy(acc_sh.at[pl.ds(s * SLAB, SLAB)],
                    out_hbm.at[c, pl.ds(s * SLAB, SLAB)])


_agg_call = pl.kernel(
    _agg_body,
    out_type=jax.ShapeDtypeStruct((NC, NPAD, HID), jnp.float32),
    mesh=_mesh,
    compiler_params=pltpu.CompilerParams(needs_layout_passes=False,
                                         use_tc_tiling_on_sc=False),
    scratch_types=[
        pltpu.VMEM((NCHUNK, CH), jnp.int32),
        pltpu.VMEM((NCHUNK, CH), jnp.int32),
        pltpu.VMEM((CH, HID), jnp.float32),
        pltpu.VMEM((CH, HID), jnp.float32),
        pltpu.VMEM_SHARED((NPAD, HID), jnp.float32),
        pltpu.SemaphoreType.DMA,
        pltpu.SemaphoreType.DMA,
        pltpu.SemaphoreType.DMA,
        pltpu.SemaphoreType.DMA,
    ],
)


def _tca_body(parts_ref, x_ref, w1_ref, hp_ref, dis_ref):
    deg = jnp.sum(parts_ref[...], axis=0) + 1.0
    dis = lax.rsqrt(deg)
    h = jnp.dot(x_ref[...], w1_ref[...], preferred_element_type=jnp.float32)
    hp_ref[...] = h * dis[:, None]
    dis_ref[...] = dis[:, None]


_tca_call = pl.pallas_call(
    _tca_body,
    grid=(NBLK,),
    in_specs=[
        pl.BlockSpec((NW, BN), lambda i: (0, i)),
        pl.BlockSpec((BN, IN_CH), lambda i: (i, 0)),
        pl.BlockSpec((IN_CH, HID), lambda i: (0, 0)),
    ],
    out_specs=[
        pl.BlockSpec((BN, HID), lambda i: (i, 0)),
        pl.BlockSpec((BN, 1), lambda i: (i, 0)),
    ],
    out_shape=[
        jax.ShapeDtypeStruct((NPAD, HID), jnp.float32),
        jax.ShapeDtypeStruct((NPAD, 1), jnp.float32),
    ],
)


def _tcb_body(acc_ref, hp_ref, dis_ref, b1_ref, w2_ref, hp2_ref):
    a = acc_ref[0] + acc_ref[1]
    h1 = jnp.maximum((a + hp_ref[...]) * dis_ref[...] + b1_ref[...], 0.0)
    h2 = jnp.dot(h1, w2_ref[...], preferred_element_type=jnp.float32)
    hp2_ref[...] = h2 * dis_ref[...]


_tcb_call = pl.pallas_call(
    _tcb_body,
    grid=(NBLK,),
    in_specs=[
        pl.BlockSpec((NC, BN, HID), lambda i: (0, i, 0)),
        pl.BlockSpec((BN, HID), lambda i: (i, 0)),
        pl.BlockSpec((BN, 1), lambda i: (i, 0)),
        pl.BlockSpec((1, HID), lambda i: (0, 0)),
        pl.BlockSpec((HID, HID), lambda i: (0, 0)),
    ],
    out_specs=pl.BlockSpec((BN, HID), lambda i: (i, 0)),
    out_shape=jax.ShapeDtypeStruct((NPAD, HID), jnp.float32),
)


def _tcc_body(acc_ref, hp2_ref, dis_ref, b2_ref, batch_ref, wl_ref, bl_ref,
              out_ref, sum_scr, cnt_scr):
    i = pl.program_id(0)

    @pl.when(i == 0)
    def _():
        sum_scr[...] = jnp.zeros_like(sum_scr)
        cnt_scr[...] = jnp.zeros_like(cnt_scr)

    a = acc_ref[0] + acc_ref[1]
    h2 = jnp.maximum((a + hp2_ref[...]) * dis_ref[...] + b2_ref[...], 0.0)
    b = batch_ref[0, 0]
    gids = lax.broadcasted_iota(jnp.int32, (G, BN), 0)
    onehot = (gids == b[None, :]).astype(jnp.float32)
    sum_scr[...] += jnp.dot(onehot, h2, preferred_element_type=jnp.float32, precision=lax.Precision.HIGHEST)
    cnt_scr[...] += jnp.dot(onehot, jnp.ones((BN, HID), jnp.float32),
                            preferred_element_type=jnp.float32, precision=lax.Precision.HIGHEST)

    @pl.when(i == NBLK - 1)
    def _():
        pooled = sum_scr[...] / jnp.maximum(cnt_scr[...], 1.0)
        out_ref[...] = (jnp.dot(pooled, wl_ref[...],
                                preferred_element_type=jnp.float32)
                        + bl_ref[...])


_tcc_call = pl.pallas_call(
    _tcc_body,
    grid=(NBLK,),
    in_specs=[
        pl.BlockSpec((NC, BN, HID), lambda i: (0, i, 0)),
        pl.BlockSpec((BN, HID), lambda i: (i, 0)),
        pl.BlockSpec((BN, 1), lambda i: (i, 0)),
        pl.BlockSpec((1, HID), lambda i: (0, 0)),
        pl.BlockSpec((1, 1, BN), lambda i: (i, 0, 0)),
        pl.BlockSpec((HID, 1), lambda i: (0, 0)),
        pl.BlockSpec((1, 1), lambda i: (0, 0)),
    ],
    out_specs=pl.BlockSpec((G, 1), lambda i: (0, 0)),
    out_shape=jax.ShapeDtypeStruct((G, 1), jnp.float32),
    scratch_shapes=[
        pltpu.VMEM((G, HID), jnp.float32),
        pltpu.VMEM((G, HID), jnp.float32),
    ],
)


@jax.jit
def kernel(x, edge_index, batch, W1, b1, W2, b2, Wl, bl):
    src = edge_index[0].astype(jnp.int32)
    dst = edge_index[1].astype(jnp.int32)
    pad_i = jnp.arange(EPAD - E, dtype=jnp.int32)
    src_p = jnp.concatenate([src, pad_i % N]).reshape(NW, NCHUNK, CH)
    dst_p = jnp.concatenate(
        [dst, N + pad_i % (NPAD - N)]).reshape(NW, NCHUNK, CH)
    x_p = jnp.concatenate([x, jnp.zeros((NPAD - N, IN_CH), jnp.float32)])
    batch_p = jnp.concatenate(
        [batch.astype(jnp.int32), jnp.full((NPAD - N,), G, jnp.int32)]
    ).reshape(NBLK, 1, BN)
    zeros_nod = jnp.zeros((NPAD, HID), jnp.float32)

    deg_parts = _deg_call(dst_p.reshape(EPAD))
    hp, dis = _tca_call(deg_parts, x_p, W1)
    acc1 = _agg_call(src_p, dst_p, hp, zeros_nod)
    hp2 = _tcb_call(acc1, hp, dis, b1.reshape(1, HID), W2)
    acc2 = _agg_call(src_p, dst_p, hp2, zeros_nod)
    out = _tcc_call(acc2, hp2, dis, b2.reshape(1, HID), batch_p,
                    Wl, bl.reshape(1, 1))
    return out


# speedup vs baseline: 39.8356x; 1.1951x over previous
import functools

import jax
import jax.numpy as jnp
from jax import lax
from jax.experimental import pallas as pl
from jax.experimental.pallas import tpu as pltpu
from jax.experimental.pallas import tpu_sc as plsc

N = 10000
E = 320000
IN_CH = 128
HID = 64
G = 128

NC = 2
NS = 16
NW = NC * NS

NPAD = 10240
EPAD = 327680
EPW = EPAD // NW
CH = 128
NCHUNK = EPW // CH
SLAB = NPAD // NS

BN = 512
NBLK = NPAD // BN

_mesh = plsc.VectorSubcoreMesh(core_axis_name="c", subcore_axis_name="s")


def _deg_body(dst_hbm, out_hbm, dst_v, deg_v):
    c = lax.axis_index("c")
    s = lax.axis_index("s")
    wid = c * NS + s

    def zero(i, carry):
        deg_v[pl.ds(i * 16, 16)] = jnp.zeros((16,), jnp.float32)
        return carry

    lax.fori_loop(0, NPAD // 16, zero, 0)

    pltpu.sync_copy(dst_hbm.at[pl.ds(wid * EPW, EPW)], dst_v)

    ones = jnp.ones((16,), jnp.float32)

    def acc(j, carry):
        dvec = dst_v[pl.ds(j * 16, 16)]
        plsc.addupdate_scatter(deg_v, [dvec], ones)
        return carry

    lax.fori_loop(0, EPW // 16, acc, 0)

    pltpu.sync_copy(deg_v, out_hbm.at[wid])


_deg_call = pl.kernel(
    _deg_body,
    out_type=jax.ShapeDtypeStruct((NW, NPAD), jnp.float32),
    mesh=_mesh,
    compiler_params=pltpu.CompilerParams(needs_layout_passes=False),
    scratch_types=[
        pltpu.VMEM((EPW,), jnp.int32),
        pltpu.VMEM((NPAD,), jnp.float32),
    ],
)


def _agg_body(src_hbm, dst_hbm, hp_hbm, zero_hbm, out_hbm,
              sidx_v, didx_v, rows0, rows1, rows2, rows3, acc_sh,
              sg0, sg1, sg2, sg3, ss0, ss1, ss2, ss3):
    c = lax.axis_index("c")
    s = lax.axis_index("s")
    wid = c * NS + s

    pltpu.sync_copy(zero_hbm.at[pl.ds(s * SLAB, SLAB)],
                    acc_sh.at[pl.ds(s * SLAB, SLAB)])
    pltpu.sync_copy(src_hbm.at[wid], sidx_v)
    pltpu.sync_copy(dst_hbm.at[wid], didx_v)
    plsc.subcore_barrier()

    rows = (rows0, rows1, rows2, rows3)
    sg = (sg0, sg1, sg2, sg3)
    ss = (ss0, ss1, ss2, ss3)
    for b in range(4):
        pltpu.async_copy(hp_hbm.at[sidx_v.at[b]], rows[b], sg[b])

    def quad(p, carry):
        k = 4 * p
        for b in range(4):
            pltpu.make_async_copy(hp_hbm.at[sidx_v.at[k]], rows[b],
                                  sg[b]).wait()
            pltpu.async_copy(rows[b], acc_sh.at[didx_v.at[k + b]], ss[b],
                             add=True)
        for b in range(4):
            pltpu.make_async_copy(rows[b], acc_sh.at[didx_v.at[k]],
                                  ss[b]).wait()
            pltpu.async_copy(hp_hbm.at[sidx_v.at[lax.rem(k + 4 + b, NCHUNK)]],
                             rows[b], sg[b])
        return carry

    lax.fori_loop(0, NCHUNK // 4, quad, 0)

    for b in range(4):
        pltpu.make_async_copy(hp_hbm.at[sidx_v.at[b]], rows[b], sg[b]).wait()

    plsc.subcore_barrier()
    pltpu.sync_copy(acc_sh.at[pl.ds(s * SLAB, SLAB)],
                    out_hbm.at[c, pl.ds(s * SLAB, SLAB)])


_agg_call = pl.kernel(
    _agg_body,
    out_type=jax.ShapeDtypeStruct((NC, NPAD, HID), jnp.float32),
    mesh=_mesh,
    compiler_params=pltpu.CompilerParams(needs_layout_passes=False,
                                         use_tc_tiling_on_sc=False),
    scratch_types=(
        [pltpu.VMEM((NCHUNK, CH), jnp.int32)] * 2
        + [pltpu.VMEM((CH, HID), jnp.float32)] * 4
        + [pltpu.VMEM_SHARED((NPAD, HID), jnp.float32)]
        + [pltpu.SemaphoreType.DMA] * 8
    ),
)


def _tca_body(parts_ref, x_ref, w1_ref, hp_ref, dis_ref):
    deg = jnp.sum(parts_ref[...], axis=0) + 1.0
    dis = lax.rsqrt(deg)
    h = jnp.dot(x_ref[...], w1_ref[...], preferred_element_type=jnp.float32)
    hp_ref[...] = h * dis[:, None]
    dis_ref[...] = dis[:, None]


_tca_call = pl.pallas_call(
    _tca_body,
    grid=(NBLK,),
    in_specs=[
        pl.BlockSpec((NW, BN), lambda i: (0, i)),
        pl.BlockSpec((BN, IN_CH), lambda i: (i, 0)),
        pl.BlockSpec((IN_CH, HID), lambda i: (0, 0)),
    ],
    out_specs=[
        pl.BlockSpec((BN, HID), lambda i: (i, 0)),
        pl.BlockSpec((BN, 1), lambda i: (i, 0)),
    ],
    out_shape=[
        jax.ShapeDtypeStruct((NPAD, HID), jnp.float32),
        jax.ShapeDtypeStruct((NPAD, 1), jnp.float32),
    ],
)


def _tcb_body(acc_ref, hp_ref, dis_ref, b1_ref, w2_ref, hp2_ref):
    a = acc_ref[0] + acc_ref[1]
    h1 = jnp.maximum((a + hp_ref[...]) * dis_ref[...] + b1_ref[...], 0.0)
    h2 = jnp.dot(h1, w2_ref[...], preferred_element_type=jnp.float32)
    hp2_ref[...] = h2 * dis_ref[...]


_tcb_call = pl.pallas_call(
    _tcb_body,
    grid=(NBLK,),
    in_specs=[
        pl.BlockSpec((NC, BN, HID), lambda i: (0, i, 0)),
        pl.BlockSpec((BN, HID), lambda i: (i, 0)),
        pl.BlockSpec((BN, 1), lambda i: (i, 0)),
        pl.BlockSpec((1, HID), lambda i: (0, 0)),
        pl.BlockSpec((HID, HID), lambda i: (0, 0)),
    ],
    out_specs=pl.BlockSpec((BN, HID), lambda i: (i, 0)),
    out_shape=jax.ShapeDtypeStruct((NPAD, HID), jnp.float32),
)


def _tcc_body(acc_ref, hp2_ref, dis_ref, b2_ref, batch_ref, wl_ref, bl_ref,
              out_ref, sum_scr, cnt_scr):
    i = pl.program_id(0)

    @pl.when(i == 0)
    def _():
        sum_scr[...] = jnp.zeros_like(sum_scr)
        cnt_scr[...] = jnp.zeros_like(cnt_scr)

    a = acc_ref[0] + acc_ref[1]
    h2 = jnp.maximum((a + hp2_ref[...]) * dis_ref[...] + b2_ref[...], 0.0)
    b = batch_ref[0, 0]
    gids = lax.broadcasted_iota(jnp.int32, (G, BN), 0)
    onehot = (gids == b[None, :]).astype(jnp.float32)
    sum_scr[...] += jnp.dot(onehot, h2, preferred_element_type=jnp.float32, precision=lax.Precision.HIGHEST)
    cnt_scr[...] += jnp.dot(onehot, jnp.ones((BN, HID), jnp.float32),
                            preferred_element_type=jnp.float32, precision=lax.Precision.HIGHEST)

    @pl.when(i == NBLK - 1)
    def _():
        pooled = sum_scr[...] / jnp.maximum(cnt_scr[...], 1.0)
        out_ref[...] = (jnp.dot(pooled, wl_ref[...],
                                preferred_element_type=jnp.float32)
                        + bl_ref[...])


_tcc_call = pl.pallas_call(
    _tcc_body,
    grid=(NBLK,),
    in_specs=[
        pl.BlockSpec((NC, BN, HID), lambda i: (0, i, 0)),
        pl.BlockSpec((BN, HID), lambda i: (i, 0)),
        pl.BlockSpec((BN, 1), lambda i: (i, 0)),
        pl.BlockSpec((1, HID), lambda i: (0, 0)),
        pl.BlockSpec((1, 1, BN), lambda i: (i, 0, 0)),
        pl.BlockSpec((HID, 1), lambda i: (0, 0)),
        pl.BlockSpec((1, 1), lambda i: (0, 0)),
    ],
    out_specs=pl.BlockSpec((G, 1), lambda i: (0, 0)),
    out_shape=jax.ShapeDtypeStruct((G, 1), jnp.float32),
    scratch_shapes=[
        pltpu.VMEM((G, HID), jnp.float32),
        pltpu.VMEM((G, HID), jnp.float32),
    ],
)


@jax.jit
def kernel(x, edge_index, batch, W1, b1, W2, b2, Wl, bl):
    src = edge_index[0].astype(jnp.int32)
    dst = edge_index[1].astype(jnp.int32)
    pad_i = jnp.arange(EPAD - E, dtype=jnp.int32)
    src_p = jnp.concatenate([src, pad_i % N]).reshape(NW, NCHUNK, CH)
    dst_p = jnp.concatenate(
        [dst, N + pad_i % (NPAD - N)]).reshape(NW, NCHUNK, CH)
    x_p = jnp.concatenate([x, jnp.zeros((NPAD - N, IN_CH), jnp.float32)])
    batch_p = jnp.concatenate(
        [batch.astype(jnp.int32), jnp.full((NPAD - N,), G, jnp.int32)]
    ).reshape(NBLK, 1, BN)
    zeros_nod = jnp.zeros((NPAD, HID), jnp.float32)

    deg_parts = _deg_call(dst_p.reshape(EPAD))
    hp, dis = _tca_call(deg_parts, x_p, W1)
    acc1 = _agg_call(src_p, dst_p, hp, zeros_nod)
    hp2 = _tcb_call(acc1, hp, dis, b1.reshape(1, HID), W2)
    acc2 = _agg_call(src_p, dst_p, hp2, zeros_nod)
    out = _tcc_call(acc2, hp2, dis, b2.reshape(1, HID), batch_p,
                    Wl, bl.reshape(1, 1))
    return out


# speedup vs baseline: 40.7546x; 1.0231x over previous
import functools

import jax
import jax.numpy as jnp
from jax import lax
from jax.experimental import pallas as pl
from jax.experimental.pallas import tpu as pltpu
from jax.experimental.pallas import tpu_sc as plsc

N = 10000
E = 320000
IN_CH = 128
HID = 64
G = 128

NC = 2
NS = 16
NW = NC * NS

NPAD = 10240
EPAD = 327680
EPW = EPAD // NW
CH = 128
NCHUNK = EPW // CH
SLAB = NPAD // NS

BN = 512
NBLK = NPAD // BN
NBUF = 8

_mesh = plsc.VectorSubcoreMesh(core_axis_name="c", subcore_axis_name="s")


def _deg_body(dst_hbm, out_hbm, dst_v, deg_v):
    c = lax.axis_index("c")
    s = lax.axis_index("s")
    wid = c * NS + s

    def zero(i, carry):
        deg_v[pl.ds(i * 16, 16)] = jnp.zeros((16,), jnp.float32)
        return carry

    lax.fori_loop(0, NPAD // 16, zero, 0)

    pltpu.sync_copy(dst_hbm.at[pl.ds(wid * EPW, EPW)], dst_v)

    ones = jnp.ones((16,), jnp.float32)

    def acc(j, carry):
        dvec = dst_v[pl.ds(j * 16, 16)]
        plsc.addupdate_scatter(deg_v, [dvec], ones)
        return carry

    lax.fori_loop(0, EPW // 16, acc, 0)

    pltpu.sync_copy(deg_v, out_hbm.at[wid])


_deg_call = pl.kernel(
    _deg_body,
    out_type=jax.ShapeDtypeStruct((NW, NPAD), jnp.float32),
    mesh=_mesh,
    compiler_params=pltpu.CompilerParams(needs_layout_passes=False),
    scratch_types=[
        pltpu.VMEM((EPW,), jnp.int32),
        pltpu.VMEM((NPAD,), jnp.float32),
    ],
)


def _agg_body(src_hbm, dst_hbm, hp_hbm, zero_hbm, out_hbm,
              sidx_v, didx_v, *scr):
    rows = scr[:NBUF]
    sg = scr[NBUF + 1:2 * NBUF + 1]
    ss = scr[2 * NBUF + 1:]
    acc_sh = scr[NBUF]
    c = lax.axis_index("c")
    s = lax.axis_index("s")
    wid = c * NS + s

    pltpu.sync_copy(zero_hbm.at[pl.ds(s * SLAB, SLAB)],
                    acc_sh.at[pl.ds(s * SLAB, SLAB)])
    pltpu.sync_copy(src_hbm.at[wid], sidx_v)
    pltpu.sync_copy(dst_hbm.at[wid], didx_v)
    plsc.subcore_barrier()

    for b in range(NBUF):
        pltpu.async_copy(hp_hbm.at[sidx_v.at[b]], rows[b], sg[b])

    def group(p, carry):
        k = NBUF * p
        for b in range(NBUF):
            pltpu.make_async_copy(hp_hbm.at[sidx_v.at[k]], rows[b],
                                  sg[b]).wait()
            pltpu.async_copy(rows[b], acc_sh.at[didx_v.at[k + b]], ss[b],
                             add=True)
        for b in range(NBUF):
            pltpu.make_async_copy(rows[b], acc_sh.at[didx_v.at[k]],
                                  ss[b]).wait()
            pltpu.async_copy(
                hp_hbm.at[sidx_v.at[lax.rem(k + NBUF + b, NCHUNK)]],
                rows[b], sg[b])
        return carry

    lax.fori_loop(0, NCHUNK // NBUF, group, 0)

    for b in range(NBUF):
        pltpu.make_async_copy(hp_hbm.at[sidx_v.at[b]], rows[b], sg[b]).wait()

    plsc.subcore_barrier()
    pltpu.sync_copy(acc_sh.at[pl.ds(s * SLAB, SLAB)],
                    out_hbm.at[c, pl.ds(s * SLAB, SLAB)])


_agg_call = pl.kernel(
    _agg_body,
    out_type=jax.ShapeDtypeStruct((NC, NPAD, HID), jnp.float32),
    mesh=_mesh,
    compiler_params=pltpu.CompilerParams(needs_layout_passes=False,
                                         use_tc_tiling_on_sc=False),
    scratch_types=(
        [pltpu.VMEM((NCHUNK, CH), jnp.int32)] * 2
        + [pltpu.VMEM((CH, HID), jnp.float32)] * NBUF
        + [pltpu.VMEM_SHARED((NPAD, HID), jnp.float32)]
        + [pltpu.SemaphoreType.DMA] * (2 * NBUF)
    ),
)


def _tca_body(parts_ref, x_ref, w1_ref, hp_ref, dis_ref):
    deg = jnp.sum(parts_ref[...], axis=0) + 1.0
    dis = lax.rsqrt(deg)
    h = jnp.dot(x_ref[...], w1_ref[...], preferred_element_type=jnp.float32)
    hp_ref[...] = h * dis[:, None]
    dis_ref[...] = dis[:, None]


_tca_call = pl.pallas_call(
    _tca_body,
    grid=(NBLK,),
    in_specs=[
        pl.BlockSpec((NW, BN), lambda i: (0, i)),
        pl.BlockSpec((BN, IN_CH), lambda i: (i, 0)),
        pl.BlockSpec((IN_CH, HID), lambda i: (0, 0)),
    ],
    out_specs=[
        pl.BlockSpec((BN, HID), lambda i: (i, 0)),
        pl.BlockSpec((BN, 1), lambda i: (i, 0)),
    ],
    out_shape=[
        jax.ShapeDtypeStruct((NPAD, HID), jnp.float32),
        jax.ShapeDtypeStruct((NPAD, 1), jnp.float32),
    ],
)


def _tcb_body(acc_ref, hp_ref, dis_ref, b1_ref, w2_ref, hp2_ref):
    a = acc_ref[0] + acc_ref[1]
    h1 = jnp.maximum((a + hp_ref[...]) * dis_ref[...] + b1_ref[...], 0.0)
    h2 = jnp.dot(h1, w2_ref[...], preferred_element_type=jnp.float32)
    hp2_ref[...] = h2 * dis_ref[...]


_tcb_call = pl.pallas_call(
    _tcb_body,
    grid=(NBLK,),
    in_specs=[
        pl.BlockSpec((NC, BN, HID), lambda i: (0, i, 0)),
        pl.BlockSpec((BN, HID), lambda i: (i, 0)),
        pl.BlockSpec((BN, 1), lambda i: (i, 0)),
        pl.BlockSpec((1, HID), lambda i: (0, 0)),
        pl.BlockSpec((HID, HID), lambda i: (0, 0)),
    ],
    out_specs=pl.BlockSpec((BN, HID), lambda i: (i, 0)),
    out_shape=jax.ShapeDtypeStruct((NPAD, HID), jnp.float32),
)


def _tcc_body(acc_ref, hp2_ref, dis_ref, b2_ref, batch_ref, wl_ref, bl_ref,
              out_ref, sum_scr, cnt_scr):
    i = pl.program_id(0)

    @pl.when(i == 0)
    def _():
        sum_scr[...] = jnp.zeros_like(sum_scr)
        cnt_scr[...] = jnp.zeros_like(cnt_scr)

    a = acc_ref[0] + acc_ref[1]
    h2 = jnp.maximum((a + hp2_ref[...]) * dis_ref[...] + b2_ref[...], 0.0)
    b = batch_ref[0, 0]
    gids = lax.broadcasted_iota(jnp.int32, (G, BN), 0)
    onehot = (gids == b[None, :]).astype(jnp.float32)
    sum_scr[...] += jnp.dot(onehot, h2, preferred_element_type=jnp.float32, precision=lax.Precision.HIGHEST)
    cnt_scr[...] += jnp.dot(onehot, jnp.ones((BN, HID), jnp.float32),
                            preferred_element_type=jnp.float32, precision=lax.Precision.HIGHEST)

    @pl.when(i == NBLK - 1)
    def _():
        pooled = sum_scr[...] / jnp.maximum(cnt_scr[...], 1.0)
        out_ref[...] = (jnp.dot(pooled, wl_ref[...],
                                preferred_element_type=jnp.float32)
                        + bl_ref[...])


_tcc_call = pl.pallas_call(
    _tcc_body,
    grid=(NBLK,),
    in_specs=[
        pl.BlockSpec((NC, BN, HID), lambda i: (0, i, 0)),
        pl.BlockSpec((BN, HID), lambda i: (i, 0)),
        pl.BlockSpec((BN, 1), lambda i: (i, 0)),
        pl.BlockSpec((1, HID), lambda i: (0, 0)),
        pl.BlockSpec((1, 1, BN), lambda i: (i, 0, 0)),
        pl.BlockSpec((HID, 1), lambda i: (0, 0)),
        pl.BlockSpec((1, 1), lambda i: (0, 0)),
    ],
    out_specs=pl.BlockSpec((G, 1), lambda i: (0, 0)),
    out_shape=jax.ShapeDtypeStruct((G, 1), jnp.float32),
    scratch_shapes=[
        pltpu.VMEM((G, HID), jnp.float32),
        pltpu.VMEM((G, HID), jnp.float32),
    ],
)


@jax.jit
def kernel(x, edge_index, batch, W1, b1, W2, b2, Wl, bl):
    src = edge_index[0].astype(jnp.int32)
    dst = edge_index[1].astype(jnp.int32)
    pad_i = jnp.arange(EPAD - E, dtype=jnp.int32)
    src_p = jnp.concatenate([src, pad_i % N]).reshape(NW, NCHUNK, CH)
    dst_p = jnp.concatenate(
        [dst, N + pad_i % (NPAD - N)]).reshape(NW, NCHUNK, CH)
    x_p = jnp.concatenate([x, jnp.zeros((NPAD - N, IN_CH), jnp.float32)])
    batch_p = jnp.concatenate(
        [batch.astype(jnp.int32), jnp.full((NPAD - N,), G, jnp.int32)]
    ).reshape(NBLK, 1, BN)
    zeros_nod = jnp.zeros((NPAD, HID), jnp.float32)

    deg_parts = _deg_call(dst_p.reshape(EPAD))
    hp, dis = _tca_call(deg_parts, x_p, W1)
    acc1 = _agg_call(src_p, dst_p, hp, zeros_nod)
    hp2 = _tcb_call(acc1, hp, dis, b1.reshape(1, HID), W2)
    acc2 = _agg_call(src_p, dst_p, hp2, zeros_nod)
    out = _tcc_call(acc2, hp2, dis, b2.reshape(1, HID), batch_p,
                    Wl, bl.reshape(1, 1))
    return out


# speedup vs baseline: 40.8425x; 1.0022x over previous
import functools

import jax
import jax.numpy as jnp
from jax import lax
from jax.experimental import pallas as pl
from jax.experimental.pallas import tpu as pltpu
from jax.experimental.pallas import tpu_sc as plsc

N = 10000
E = 320000
IN_CH = 128
HID = 64
G = 128

NC = 2
NS = 16
NW = NC * NS

NPAD = 10240
EPAD = 327680
EPW = EPAD // NW
CH = 128
NCHUNK = EPW // CH
SLAB = NPAD // NS

BN = 512
NBLK = NPAD // BN
NBUF = 8

_mesh = plsc.VectorSubcoreMesh(core_axis_name="c", subcore_axis_name="s")


def _deg_body(dst_hbm, out_hbm, dst_v, deg_v):
    c = lax.axis_index("c")
    s = lax.axis_index("s")
    wid = c * NS + s

    def zero(i, carry):
        deg_v[pl.ds(i * 16, 16)] = jnp.zeros((16,), jnp.float32)
        return carry

    lax.fori_loop(0, NPAD // 16, zero, 0)

    pltpu.sync_copy(dst_hbm.at[pl.ds(wid * EPW, EPW)], dst_v)

    ones = jnp.ones((16,), jnp.float32)

    def acc(j, carry):
        dvec = dst_v[pl.ds(j * 16, 16)]
        plsc.addupdate_scatter(deg_v, [dvec], ones)
        return carry

    lax.fori_loop(0, EPW // 16, acc, 0)

    pltpu.sync_copy(deg_v, out_hbm.at[wid])


_deg_call = pl.kernel(
    _deg_body,
    out_type=jax.ShapeDtypeStruct((NW, NPAD), jnp.float32),
    mesh=_mesh,
    compiler_params=pltpu.CompilerParams(needs_layout_passes=False,
                                         skip_device_barrier=True),
    scratch_types=[
        pltpu.VMEM((EPW,), jnp.int32),
        pltpu.VMEM((NPAD,), jnp.float32),
    ],
)


def _agg_body(src_hbm, dst_hbm, hp_hbm, zero_hbm, out_hbm,
              sidx_v, didx_v, *scr):
    rows = scr[:NBUF]
    sg = scr[NBUF + 1:2 * NBUF + 1]
    ss = scr[2 * NBUF + 1:]
    acc_sh = scr[NBUF]
    c = lax.axis_index("c")
    s = lax.axis_index("s")
    wid = c * NS + s

    pltpu.sync_copy(zero_hbm.at[pl.ds(s * SLAB, SLAB)],
                    acc_sh.at[pl.ds(s * SLAB, SLAB)])
    pltpu.sync_copy(src_hbm.at[wid], sidx_v)
    pltpu.sync_copy(dst_hbm.at[wid], didx_v)
    plsc.subcore_barrier()

    for b in range(NBUF):
        pltpu.async_copy(hp_hbm.at[sidx_v.at[b]], rows[b], sg[b])

    def group(p, carry):
        k = NBUF * p
        for b in range(NBUF):
            pltpu.make_async_copy(hp_hbm.at[sidx_v.at[k]], rows[b],
                                  sg[b]).wait()
            pltpu.async_copy(rows[b], acc_sh.at[didx_v.at[k + b]], ss[b],
                             add=True)
        for b in range(NBUF):
            pltpu.make_async_copy(rows[b], acc_sh.at[didx_v.at[k]],
                                  ss[b]).wait()
            pltpu.async_copy(
                hp_hbm.at[sidx_v.at[lax.rem(k + NBUF + b, NCHUNK)]],
                rows[b], sg[b])
        return carry

    lax.fori_loop(0, NCHUNK // NBUF, group, 0)

    for b in range(NBUF):
        pltpu.make_async_copy(hp_hbm.at[sidx_v.at[b]], rows[b], sg[b]).wait()

    plsc.subcore_barrier()
    pltpu.sync_copy(acc_sh.at[pl.ds(s * SLAB, SLAB)],
                    out_hbm.at[c, pl.ds(s * SLAB, SLAB)])


_agg_call = pl.kernel(
    _agg_body,
    out_type=jax.ShapeDtypeStruct((NC, NPAD, HID), jnp.float32),
    mesh=_mesh,
    compiler_params=pltpu.CompilerParams(needs_layout_passes=False,
                                         use_tc_tiling_on_sc=False,
                                         skip_device_barrier=True),
    scratch_types=(
        [pltpu.VMEM((NCHUNK, CH), jnp.int32)] * 2
        + [pltpu.VMEM((CH, HID), jnp.float32)] * NBUF
        + [pltpu.VMEM_SHARED((NPAD, HID), jnp.float32)]
        + [pltpu.SemaphoreType.DMA] * (2 * NBUF)
    ),
)


def _tca_body(parts_ref, x_ref, w1_ref, hp_ref, dis_ref):
    deg = jnp.sum(parts_ref[...], axis=0) + 1.0
    dis = lax.rsqrt(deg)
    h = jnp.dot(x_ref[...], w1_ref[...], preferred_element_type=jnp.float32)
    hp_ref[...] = h * dis[:, None]
    dis_ref[...] = dis[:, None]


_tca_call = pl.pallas_call(
    _tca_body,
    grid=(NBLK,),
    in_specs=[
        pl.BlockSpec((NW, BN), lambda i: (0, i)),
        pl.BlockSpec((BN, IN_CH), lambda i: (i, 0)),
        pl.BlockSpec((IN_CH, HID), lambda i: (0, 0)),
    ],
    out_specs=[
        pl.BlockSpec((BN, HID), lambda i: (i, 0)),
        pl.BlockSpec((BN, 1), lambda i: (i, 0)),
    ],
    out_shape=[
        jax.ShapeDtypeStruct((NPAD, HID), jnp.float32),
        jax.ShapeDtypeStruct((NPAD, 1), jnp.float32),
    ],
    compiler_params=pltpu.CompilerParams(skip_device_barrier=True),
)


def _tcb_body(acc_ref, hp_ref, dis_ref, b1_ref, w2_ref, hp2_ref):
    a = acc_ref[0] + acc_ref[1]
    h1 = jnp.maximum((a + hp_ref[...]) * dis_ref[...] + b1_ref[...], 0.0)
    h2 = jnp.dot(h1, w2_ref[...], preferred_element_type=jnp.float32)
    hp2_ref[...] = h2 * dis_ref[...]


_tcb_call = pl.pallas_call(
    _tcb_body,
    grid=(NBLK,),
    in_specs=[
        pl.BlockSpec((NC, BN, HID), lambda i: (0, i, 0)),
        pl.BlockSpec((BN, HID), lambda i: (i, 0)),
        pl.BlockSpec((BN, 1), lambda i: (i, 0)),
        pl.BlockSpec((1, HID), lambda i: (0, 0)),
        pl.BlockSpec((HID, HID), lambda i: (0, 0)),
    ],
    out_specs=pl.BlockSpec((BN, HID), lambda i: (i, 0)),
    out_shape=jax.ShapeDtypeStruct((NPAD, HID), jnp.float32),
    compiler_params=pltpu.CompilerParams(skip_device_barrier=True),
)


def _tcc_body(acc_ref, hp2_ref, dis_ref, b2_ref, batch_ref, wl_ref, bl_ref,
              out_ref, sum_scr, cnt_scr):
    i = pl.program_id(0)

    @pl.when(i == 0)
    def _():
        sum_scr[...] = jnp.zeros_like(sum_scr)
        cnt_scr[...] = jnp.zeros_like(cnt_scr)

    a = acc_ref[0] + acc_ref[1]
    h2 = jnp.maximum((a + hp2_ref[...]) * dis_ref[...] + b2_ref[...], 0.0)
    b = batch_ref[0, 0]
    gids = lax.broadcasted_iota(jnp.int32, (G, BN), 0)
    onehot = (gids == b[None, :]).astype(jnp.float32)
    sum_scr[...] += jnp.dot(onehot, h2, preferred_element_type=jnp.float32, precision=lax.Precision.HIGHEST)
    cnt_scr[...] += jnp.dot(onehot, jnp.ones((BN, HID), jnp.float32),
                            preferred_element_type=jnp.float32, precision=lax.Precision.HIGHEST)

    @pl.when(i == NBLK - 1)
    def _():
        pooled = sum_scr[...] / jnp.maximum(cnt_scr[...], 1.0)
        out_ref[...] = (jnp.dot(pooled, wl_ref[...],
                                preferred_element_type=jnp.float32)
                        + bl_ref[...])


_tcc_call = pl.pallas_call(
    _tcc_body,
    grid=(NBLK,),
    in_specs=[
        pl.BlockSpec((NC, BN, HID), lambda i: (0, i, 0)),
        pl.BlockSpec((BN, HID), lambda i: (i, 0)),
        pl.BlockSpec((BN, 1), lambda i: (i, 0)),
        pl.BlockSpec((1, HID), lambda i: (0, 0)),
        pl.BlockSpec((1, 1, BN), lambda i: (i, 0, 0)),
        pl.BlockSpec((HID, 1), lambda i: (0, 0)),
        pl.BlockSpec((1, 1), lambda i: (0, 0)),
    ],
    out_specs=pl.BlockSpec((G, 1), lambda i: (0, 0)),
    out_shape=jax.ShapeDtypeStruct((G, 1), jnp.float32),
    scratch_shapes=[
        pltpu.VMEM((G, HID), jnp.float32),
        pltpu.VMEM((G, HID), jnp.float32),
    ],
    compiler_params=pltpu.CompilerParams(skip_device_barrier=True),
)


@jax.jit
def kernel(x, edge_index, batch, W1, b1, W2, b2, Wl, bl):
    src = edge_index[0].astype(jnp.int32)
    dst = edge_index[1].astype(jnp.int32)
    pad_i = jnp.arange(EPAD - E, dtype=jnp.int32)
    src_p = jnp.concatenate([src, pad_i % N]).reshape(NW, NCHUNK, CH)
    dst_p = jnp.concatenate(
        [dst, N + pad_i % (NPAD - N)]).reshape(NW, NCHUNK, CH)
    x_p = jnp.concatenate([x, jnp.zeros((NPAD - N, IN_CH), jnp.float32)])
    batch_p = jnp.concatenate(
        [batch.astype(jnp.int32), jnp.full((NPAD - N,), G, jnp.int32)]
    ).reshape(NBLK, 1, BN)
    zeros_nod = jnp.zeros((NPAD, HID), jnp.float32)

    deg_parts = _deg_call(dst_p.reshape(EPAD))
    hp, dis = _tca_call(deg_parts, x_p, W1)
    acc1 = _agg_call(src_p, dst_p, hp, zeros_nod)
    hp2 = _tcb_call(acc1, hp, dis, b1.reshape(1, HID), W2)
    acc2 = _agg_call(src_p, dst_p, hp2, zeros_nod)
    out = _tcc_call(acc2, hp2, dis, b2.reshape(1, HID), batch_p,
                    Wl, bl.reshape(1, 1))
    return out


# speedup vs baseline: 41.9628x; 1.0274x over previous
import functools

import jax
import jax.numpy as jnp
from jax import lax
from jax.experimental import pallas as pl
from jax.experimental.pallas import tpu as pltpu
from jax.experimental.pallas import tpu_sc as plsc

N = 10000
E = 320000
IN_CH = 128
HID = 64
G = 128

NC = 2
NS = 16
NW = NC * NS

NPAD = 10240
EPAD = 327680
EPW = EPAD // NW
CH = 128
NCHUNK = EPW // CH
SLAB = NPAD // NS

BN = 512
NBLK = NPAD // BN
NBUF = 8

_mesh = plsc.VectorSubcoreMesh(core_axis_name="c", subcore_axis_name="s")


def _deg_body(ei_hbm, out_hbm, dst_v, deg_v):
    c = lax.axis_index("c")
    s = lax.axis_index("s")
    wid = c * NS + s

    def zero(i, carry):
        deg_v[pl.ds(i * 16, 16)] = jnp.zeros((16,), jnp.float32)
        return carry

    lax.fori_loop(0, NPAD // 16, zero, 0)

    pltpu.sync_copy(ei_hbm.at[1, wid], dst_v)

    ones = jnp.ones((16,), jnp.float32)

    def acc(j, carry):
        dvec = dst_v[j // 8, pl.ds((j % 8) * 16, 16)]
        plsc.addupdate_scatter(deg_v, [dvec], ones)
        return carry

    lax.fori_loop(0, EPW // 16, acc, 0)

    pltpu.sync_copy(deg_v, out_hbm.at[wid])


_deg_call = pl.kernel(
    _deg_body,
    out_type=jax.ShapeDtypeStruct((NW, NPAD), jnp.float32),
    mesh=_mesh,
    compiler_params=pltpu.CompilerParams(needs_layout_passes=False,
                                         skip_device_barrier=True),
    scratch_types=[
        pltpu.VMEM((NCHUNK, CH), jnp.int32),
        pltpu.VMEM((NPAD,), jnp.float32),
    ],
)


def _agg_body(ei_hbm, hp_hbm, out_hbm, sidx_v, didx_v, *scr):
    rows = scr[:NBUF]
    sg = scr[NBUF + 1:2 * NBUF + 1]
    ss = scr[2 * NBUF + 1:]
    acc_sh = scr[NBUF]
    c = lax.axis_index("c")
    s = lax.axis_index("s")
    wid = c * NS + s

    pltpu.sync_copy(ei_hbm.at[0, wid], sidx_v)
    pltpu.sync_copy(ei_hbm.at[1, wid], didx_v)

    def zrow(i, carry):
        rows[0][i // 4, pl.ds((i % 4) * 16, 16)] = jnp.zeros(
            (16,), jnp.float32)
        return carry

    lax.fori_loop(0, CH * HID // 16, zrow, 0)
    for i in range(SLAB // CH):
        pltpu.sync_copy(rows[0], acc_sh.at[pl.ds(s * SLAB + i * CH, CH)])
    plsc.subcore_barrier()

    for b in range(NBUF):
        pltpu.async_copy(hp_hbm.at[sidx_v.at[b]], rows[b], sg[b])

    def group(p, carry):
        k = NBUF * p
        for b in range(NBUF):
            pltpu.make_async_copy(hp_hbm.at[sidx_v.at[k]], rows[b],
                                  sg[b]).wait()
            pltpu.async_copy(rows[b], acc_sh.at[didx_v.at[k + b]], ss[b],
                             add=True)
        for b in range(NBUF):
            pltpu.make_async_copy(rows[b], acc_sh.at[didx_v.at[k]],
                                  ss[b]).wait()
            pltpu.async_copy(
                hp_hbm.at[sidx_v.at[lax.rem(k + NBUF + b, NCHUNK)]],
                rows[b], sg[b])
        return carry

    lax.fori_loop(0, NCHUNK // NBUF, group, 0)

    for b in range(NBUF):
        pltpu.make_async_copy(hp_hbm.at[sidx_v.at[b]], rows[b], sg[b]).wait()

    plsc.subcore_barrier()
    pltpu.sync_copy(acc_sh.at[pl.ds(s * SLAB, SLAB)],
                    out_hbm.at[c, pl.ds(s * SLAB, SLAB)])


_agg_call = pl.kernel(
    _agg_body,
    out_type=jax.ShapeDtypeStruct((NC, NPAD, HID), jnp.float32),
    mesh=_mesh,
    compiler_params=pltpu.CompilerParams(needs_layout_passes=False,
                                         use_tc_tiling_on_sc=False,
                                         skip_device_barrier=True),
    scratch_types=(
        [pltpu.VMEM((NCHUNK, CH), jnp.int32)] * 2
        + [pltpu.VMEM((CH, HID), jnp.float32)] * NBUF
        + [pltpu.VMEM_SHARED((NPAD, HID), jnp.float32)]
        + [pltpu.SemaphoreType.DMA] * (2 * NBUF)
    ),
)


def _tca_body(parts_ref, x_ref, w1_ref, hp_ref, dis_ref):
    deg = jnp.sum(parts_ref[...], axis=0) + 1.0
    dis = lax.rsqrt(deg)
    h = jnp.dot(x_ref[...], w1_ref[...], preferred_element_type=jnp.float32)
    hp_ref[...] = h * dis[:, None]
    dis_ref[...] = dis[:, None]


_tca_call = pl.pallas_call(
    _tca_body,
    grid=(NBLK,),
    in_specs=[
        pl.BlockSpec((NW, BN), lambda i: (0, i)),
        pl.BlockSpec((BN, IN_CH), lambda i: (i, 0)),
        pl.BlockSpec((IN_CH, HID), lambda i: (0, 0)),
    ],
    out_specs=[
        pl.BlockSpec((BN, HID), lambda i: (i, 0)),
        pl.BlockSpec((BN, 1), lambda i: (i, 0)),
    ],
    out_shape=[
        jax.ShapeDtypeStruct((NPAD, HID), jnp.float32),
        jax.ShapeDtypeStruct((NPAD, 1), jnp.float32),
    ],
    compiler_params=pltpu.CompilerParams(skip_device_barrier=True),
)


def _tcb_body(acc_ref, hp_ref, dis_ref, b1_ref, w2_ref, hp2_ref):
    a = acc_ref[0] + acc_ref[1]
    h1 = jnp.maximum((a + hp_ref[...]) * dis_ref[...] + b1_ref[...], 0.0)
    h2 = jnp.dot(h1, w2_ref[...], preferred_element_type=jnp.float32)
    hp2_ref[...] = h2 * dis_ref[...]


_tcb_call = pl.pallas_call(
    _tcb_body,
    grid=(NBLK,),
    in_specs=[
        pl.BlockSpec((NC, BN, HID), lambda i: (0, i, 0)),
        pl.BlockSpec((BN, HID), lambda i: (i, 0)),
        pl.BlockSpec((BN, 1), lambda i: (i, 0)),
        pl.BlockSpec((1, HID), lambda i: (0, 0)),
        pl.BlockSpec((HID, HID), lambda i: (0, 0)),
    ],
    out_specs=pl.BlockSpec((BN, HID), lambda i: (i, 0)),
    out_shape=jax.ShapeDtypeStruct((NPAD, HID), jnp.float32),
    compiler_params=pltpu.CompilerParams(skip_device_barrier=True),
)


def _tcc_body(acc_ref, hp2_ref, dis_ref, b2_ref, batch_ref, wl_ref, bl_ref,
              out_ref, sum_scr, cnt_scr):
    i = pl.program_id(0)

    @pl.when(i == 0)
    def _():
        sum_scr[...] = jnp.zeros_like(sum_scr)
        cnt_scr[...] = jnp.zeros_like(cnt_scr)

    a = acc_ref[0] + acc_ref[1]
    h2 = jnp.maximum((a + hp2_ref[...]) * dis_ref[...] + b2_ref[...], 0.0)
    b = batch_ref[0, 0]
    gids = lax.broadcasted_iota(jnp.int32, (G, BN), 0)
    onehot = (gids == b[None, :]).astype(jnp.float32)
    sum_scr[...] += jnp.dot(onehot, h2, preferred_element_type=jnp.float32, precision=lax.Precision.HIGHEST)
    cnt_scr[...] += jnp.dot(onehot, jnp.ones((BN, HID), jnp.float32),
                            preferred_element_type=jnp.float32, precision=lax.Precision.HIGHEST)

    @pl.when(i == NBLK - 1)
    def _():
        pooled = sum_scr[...] / jnp.maximum(cnt_scr[...], 1.0)
        out_ref[...] = (jnp.dot(pooled, wl_ref[...],
                                preferred_element_type=jnp.float32)
                        + bl_ref[...])


_tcc_call = pl.pallas_call(
    _tcc_body,
    grid=(NBLK,),
    in_specs=[
        pl.BlockSpec((NC, BN, HID), lambda i: (0, i, 0)),
        pl.BlockSpec((BN, HID), lambda i: (i, 0)),
        pl.BlockSpec((BN, 1), lambda i: (i, 0)),
        pl.BlockSpec((1, HID), lambda i: (0, 0)),
        pl.BlockSpec((1, 1, BN), lambda i: (i, 0, 0)),
        pl.BlockSpec((HID, 1), lambda i: (0, 0)),
        pl.BlockSpec((1, 1), lambda i: (0, 0)),
    ],
    out_specs=pl.BlockSpec((G, 1), lambda i: (0, 0)),
    out_shape=jax.ShapeDtypeStruct((G, 1), jnp.float32),
    scratch_shapes=[
        pltpu.VMEM((G, HID), jnp.float32),
        pltpu.VMEM((G, HID), jnp.float32),
    ],
    compiler_params=pltpu.CompilerParams(skip_device_barrier=True),
)


@jax.jit
def kernel(x, edge_index, batch, W1, b1, W2, b2, Wl, bl):
    pad_i = jnp.arange(EPAD - E, dtype=jnp.int32)
    ei_p = jnp.concatenate(
        [edge_index.astype(jnp.int32),
         jnp.stack([pad_i % N, N + pad_i % (NPAD - N)])],
        axis=1).reshape(2, NW, NCHUNK, CH)
    x_p = jnp.concatenate([x, jnp.zeros((NPAD - N, IN_CH), jnp.float32)])
    batch_p = jnp.concatenate(
        [batch.astype(jnp.int32), jnp.full((NPAD - N,), G, jnp.int32)]
    ).reshape(NBLK, 1, BN)

    deg_parts = _deg_call(ei_p)
    hp, dis = _tca_call(deg_parts, x_p, W1)
    acc1 = _agg_call(ei_p, hp)
    hp2 = _tcb_call(acc1, hp, dis, b1.reshape(1, HID), W2)
    acc2 = _agg_call(ei_p, hp2)
    out = _tcc_call(acc2, hp2, dis, b2.reshape(1, HID), batch_p,
                    Wl, bl.reshape(1, 1))
    return out
